# Initial kernel scaffold; baseline (speedup 1.0000x reference)
import jax
import jax.numpy as jnp
from jax.experimental import pallas as pl


def kernel(x, edge_index, edge_label_index, W1, b1, W2, b2):
    raise NotImplementedError("write your pallas kernel here")


# speedup vs baseline: 2.7403x; 2.7403x over previous
import functools

import jax
import jax.numpy as jnp
from jax.experimental import pallas as pl

N = 10000
D_IN = 128
D_H = 256
D_OUT = 128
ROW_BLK = 2000


def _mm_body(x_ref, w_ref, o_ref):
    o_ref[...] = jnp.dot(x_ref[...], w_ref[...], preferred_element_type=jnp.float32)


def _tc_matmul(x, w):
    n, k = x.shape
    m = w.shape[1]
    return pl.pallas_call(
        _mm_body,
        grid=(n // ROW_BLK,),
        in_specs=[
            pl.BlockSpec((ROW_BLK, k), lambda i: (i, 0)),
            pl.BlockSpec((k, m), lambda i: (0, 0)),
        ],
        out_specs=pl.BlockSpec((ROW_BLK, m), lambda i: (i, 0)),
        out_shape=jax.ShapeDtypeStruct((n, m), jnp.float32),
    )(x, w)


def _scale_body(h_ref, deg_ref, hs_ref, dinv_ref):
    dinv = jax.lax.rsqrt(deg_ref[...])
    dinv_ref[...] = dinv
    hs_ref[...] = h_ref[...] * dinv


def _tc_scale(h, deg2d):
    n, m = h.shape
    return pl.pallas_call(
        _scale_body,
        grid=(n // ROW_BLK,),
        in_specs=[
            pl.BlockSpec((ROW_BLK, m), lambda i: (i, 0)),
            pl.BlockSpec((ROW_BLK, 1), lambda i: (i, 0)),
        ],
        out_specs=[
            pl.BlockSpec((ROW_BLK, m), lambda i: (i, 0)),
            pl.BlockSpec((ROW_BLK, 1), lambda i: (i, 0)),
        ],
        out_shape=[
            jax.ShapeDtypeStruct((n, m), jnp.float32),
            jax.ShapeDtypeStruct((n, 1), jnp.float32),
        ],
    )(h, deg2d)


def _mid_body(agg_ref, hs1_ref, dinv_ref, b1_ref, w2_ref, hs2_ref):
    out1 = jax.nn.relu(dinv_ref[...] * (agg_ref[...] + hs1_ref[...]) + b1_ref[...])
    h2 = jnp.dot(out1, w2_ref[...], preferred_element_type=jnp.float32)
    hs2_ref[...] = h2 * dinv_ref[...]


def _tc_mid(agg1, hs1, dinv, b1, W2):
    n = agg1.shape[0]
    return pl.pallas_call(
        _mid_body,
        grid=(n // ROW_BLK,),
        in_specs=[
            pl.BlockSpec((ROW_BLK, D_H), lambda i: (i, 0)),
            pl.BlockSpec((ROW_BLK, D_H), lambda i: (i, 0)),
            pl.BlockSpec((ROW_BLK, 1), lambda i: (i, 0)),
            pl.BlockSpec((1, D_H), lambda i: (0, 0)),
            pl.BlockSpec((D_H, D_OUT), lambda i: (0, 0)),
        ],
        out_specs=pl.BlockSpec((ROW_BLK, D_OUT), lambda i: (i, 0)),
        out_shape=jax.ShapeDtypeStruct((n, D_OUT), jnp.float32),
    )(agg1, hs1, dinv, b1, W2)


def _z_body(agg_ref, hs2_ref, dinv_ref, b2_ref, z_ref):
    z_ref[...] = dinv_ref[...] * (agg_ref[...] + hs2_ref[...]) + b2_ref[...]


def _tc_z(agg2, hs2, dinv, b2):
    n = agg2.shape[0]
    return pl.pallas_call(
        _z_body,
        grid=(n // ROW_BLK,),
        in_specs=[
            pl.BlockSpec((ROW_BLK, D_OUT), lambda i: (i, 0)),
            pl.BlockSpec((ROW_BLK, D_OUT), lambda i: (i, 0)),
            pl.BlockSpec((ROW_BLK, 1), lambda i: (i, 0)),
            pl.BlockSpec((1, D_OUT), lambda i: (0, 0)),
        ],
        out_specs=pl.BlockSpec((ROW_BLK, D_OUT), lambda i: (i, 0)),
        out_shape=jax.ShapeDtypeStruct((n, D_OUT), jnp.float32),
    )(agg2, hs2, dinv, b2)


def _dot_body(zs_ref, zd_ref, o_ref):
    o_ref[...] = jnp.sum(zs_ref[...] * zd_ref[...], axis=-1, keepdims=True)


def _tc_dot(zs, zd):
    n = zs.shape[0]
    blk = 2000
    return pl.pallas_call(
        _dot_body,
        grid=(n // blk,),
        in_specs=[
            pl.BlockSpec((blk, D_OUT), lambda i: (i, 0)),
            pl.BlockSpec((blk, D_OUT), lambda i: (i, 0)),
        ],
        out_specs=pl.BlockSpec((blk, 1), lambda i: (i, 0)),
        out_shape=jax.ShapeDtypeStruct((n, 1), jnp.float32),
    )(zs, zd)


def kernel(x, edge_index, edge_label_index, W1, b1, W2, b2):
    src = edge_index[0]
    dst = edge_index[1]

    deg = jnp.zeros((N,), jnp.float32).at[dst].add(1.0) + 1.0

    h1 = _tc_matmul(x, W1)
    hs1, dinv = _tc_scale(h1, deg[:, None])

    agg1 = jnp.zeros((N, D_H), jnp.float32).at[dst].add(hs1[src])
    hs2 = _tc_mid(agg1, hs1, dinv, b1[None, :], W2)

    agg2 = jnp.zeros((N, D_OUT), jnp.float32).at[dst].add(hs2[src])
    z = _tc_z(agg2, hs2, dinv, b2[None, :])

    zs = z[edge_label_index[0]]
    zd = z[edge_label_index[1]]
    return _tc_dot(zs, zd)[:, 0]


# speedup vs baseline: 8.4238x; 3.0740x over previous
import jax
import jax.numpy as jnp
from jax import lax
from jax.experimental import pallas as pl
from jax.experimental.pallas import tpu as pltpu
from jax.experimental.pallas import tpu_sc as plsc

N = 10000
D_IN = 128
D_H = 256
D_OUT = 128
E = 320000
EL = 20000

NC = 2
NS = 16
W = 128

EP = 323584
NP = 10240
PAD_ROW = N
RPS = NP // NS
NWIN_HALF = EP // NS // W
NWIN_FULL = EP // (NC * NS) // W
ELP = 40960
NWIN_DEC = ELP // (NC * NS) // W

ROW_BLK = 2000

_MESH = plsc.VectorSubcoreMesh(core_axis_name="c", subcore_axis_name="s")
_f32 = jnp.float32


def _zero_acc(z_hbm, acc, buf, sid):
    @pl.loop(0, RPS // W)
    def _(k):
        r = sid * RPS + k * W
        pltpu.sync_copy(z_hbm.at[pl.ds(r, W)], buf)
        pltpu.sync_copy(buf, acc.at[pl.ds(r, W)])


def _dump_acc(acc, out_hbm, buf, sid):
    @pl.loop(0, RPS // W)
    def _(k):
        r = sid * RPS + k * W
        pltpu.sync_copy(acc.at[pl.ds(r, W)], buf)
        pltpu.sync_copy(buf, out_hbm.at[pl.ds(r, W)])


def _deg_body(dst_hbm, z_hbm, o_hbm, out_hbm, acc, ones, buf, didx, sem):
    c = lax.axis_index("c")
    sid = lax.axis_index("s")
    wid = sid * NC + c

    pltpu.sync_copy(o_hbm, ones)

    @pl.loop(0, RPS // W)
    def _(k):
        r = sid * RPS + k * W
        pltpu.sync_copy(z_hbm.at[pl.ds(r, W)], buf)
        pltpu.sync_copy(buf, acc.at[pl.ds(r, W)])

    plsc.subcore_barrier()

    base0 = wid * (EP // (NC * NS))

    @pl.loop(0, NWIN_FULL)
    def _(j):
        b = base0 + j * W
        pltpu.sync_copy(dst_hbm.at[pl.ds(b, W)], didx.at[0])
        pltpu.sync_copy(ones, acc.at[didx.at[0]], add=True)

    plsc.subcore_barrier()

    @pl.loop(0, RPS // W)
    def _(k):
        r = sid * RPS + k * W
        pltpu.sync_copy(acc.at[pl.ds(r, W)], buf)
        pltpu.sync_copy(buf, out_hbm.at[c, pl.ds(r, W)])


_deg_call = pl.kernel(
    _deg_body,
    out_type=jax.ShapeDtypeStruct((NC, NP, 128), _f32),
    mesh=_MESH,
    scratch_types=[
        pltpu.VMEM_SHARED((NP, 128), _f32),
        pltpu.VMEM((W, 128), _f32),
        pltpu.VMEM((W, 128), _f32),
        pltpu.VMEM((1, W), jnp.int32),
        pltpu.SemaphoreType.DMA,
    ],
)


def _agg_run(tab_hbm, out_hbm, src_hbm, dst_hbm, z_hbm, acc, rows, buf, sidx,
             didx, sem, sid, nwin, base0):
    _zero_acc(z_hbm, acc, buf, sid)
    plsc.subcore_barrier()

    @pl.loop(0, nwin)
    def _(j):
        b = base0 + j * W
        pltpu.sync_copy(src_hbm.at[pl.ds(b, W)], sidx.at[0])
        pltpu.sync_copy(dst_hbm.at[pl.ds(b, W)], didx.at[0])
        pltpu.async_copy(tab_hbm.at[sidx.at[0]], rows, sem).wait()
        pltpu.sync_copy(rows, acc.at[didx.at[0]], add=True)

    plsc.subcore_barrier()
    _dump_acc(acc, out_hbm, buf, sid)


def _agg1_body(tab_a, tab_b, src_hbm, dst_hbm, z_hbm, out_a, out_b, acc, rows,
               buf, sidx, didx, sem):
    c = lax.axis_index("c")
    sid = lax.axis_index("s")
    base0 = sid * (EP // NS)

    @pl.when(c == 0)
    def _():
        _agg_run(tab_a, out_a, src_hbm, dst_hbm, z_hbm, acc, rows, buf, sidx,
                 didx, sem, sid, NWIN_HALF, base0)

    @pl.when(c == 1)
    def _():
        _agg_run(tab_b, out_b, src_hbm, dst_hbm, z_hbm, acc, rows, buf, sidx,
                 didx, sem, sid, NWIN_HALF, base0)


_agg1_call = pl.kernel(
    _agg1_body,
    out_type=[
        jax.ShapeDtypeStruct((NP, 128), _f32),
        jax.ShapeDtypeStruct((NP, 128), _f32),
    ],
    mesh=_MESH,
    scratch_types=[
        pltpu.VMEM_SHARED((NP, 128), _f32),
        pltpu.VMEM((W, 128), _f32),
        pltpu.VMEM((W, 128), _f32),
        pltpu.VMEM((1, W), jnp.int32),
        pltpu.VMEM((1, W), jnp.int32),
        pltpu.SemaphoreType.DMA,
    ],
)


def _agg2_body(tab_hbm, src_hbm, dst_hbm, z_hbm, out_hbm, acc, rows, buf, sidx,
               didx, sem):
    c = lax.axis_index("c")
    sid = lax.axis_index("s")
    wid = sid * NC + c

    _zero_acc(z_hbm, acc, buf, sid)
    plsc.subcore_barrier()

    base0 = wid * (EP // (NC * NS))

    @pl.loop(0, NWIN_FULL)
    def _(j):
        b = base0 + j * W
        pltpu.sync_copy(src_hbm.at[pl.ds(b, W)], sidx.at[0])
        pltpu.sync_copy(dst_hbm.at[pl.ds(b, W)], didx.at[0])
        pltpu.async_copy(tab_hbm.at[sidx.at[0]], rows, sem).wait()
        pltpu.sync_copy(rows, acc.at[didx.at[0]], add=True)

    plsc.subcore_barrier()

    @pl.loop(0, RPS // W)
    def _(k):
        r = sid * RPS + k * W
        pltpu.sync_copy(acc.at[pl.ds(r, W)], buf)
        pltpu.sync_copy(buf, out_hbm.at[c, pl.ds(r, W)])


_agg2_call = pl.kernel(
    _agg2_body,
    out_type=jax.ShapeDtypeStruct((NC, NP, 128), _f32),
    mesh=_MESH,
    scratch_types=[
        pltpu.VMEM_SHARED((NP, 128), _f32),
        pltpu.VMEM((W, 128), _f32),
        pltpu.VMEM((W, 128), _f32),
        pltpu.VMEM((1, W), jnp.int32),
        pltpu.VMEM((1, W), jnp.int32),
        pltpu.SemaphoreType.DMA,
    ],
)


def _dec_body(tab_hbm, idx_hbm, out_hbm, rows, gidx, sem):
    c = lax.axis_index("c")
    sid = lax.axis_index("s")
    wid = sid * NC + c
    base0 = wid * (ELP // (NC * NS))

    @pl.loop(0, NWIN_DEC)
    def _(j):
        b = base0 + j * W
        pltpu.sync_copy(idx_hbm.at[pl.ds(b, W)], gidx.at[0])
        pltpu.async_copy(tab_hbm.at[gidx.at[0]], rows, sem).wait()
        pltpu.sync_copy(rows, out_hbm.at[pl.ds(b, W)])


_dec_call = pl.kernel(
    _dec_body,
    out_type=jax.ShapeDtypeStruct((ELP, 128), _f32),
    mesh=_MESH,
    scratch_types=[
        pltpu.VMEM((W, 128), _f32),
        pltpu.VMEM((1, W), jnp.int32),
        pltpu.SemaphoreType.DMA,
    ],
)


def _mm_body(x_ref, w_ref, o_ref):
    o_ref[...] = jnp.dot(x_ref[...], w_ref[...], preferred_element_type=jnp.float32)


def _tc_matmul(x, w):
    n, k = x.shape
    m = w.shape[1]
    return pl.pallas_call(
        _mm_body,
        grid=(n // ROW_BLK,),
        in_specs=[
            pl.BlockSpec((ROW_BLK, k), lambda i: (i, 0)),
            pl.BlockSpec((k, m), lambda i: (0, 0)),
        ],
        out_specs=pl.BlockSpec((ROW_BLK, m), lambda i: (i, 0)),
        out_shape=jax.ShapeDtypeStruct((n, m), jnp.float32),
    )(x, w)


def _scale_body(h_ref, dega_ref, degb_ref, hs_ref, dinv_ref):
    dinv = jax.lax.rsqrt(dega_ref[...] + degb_ref[...])
    dinv_ref[...] = dinv
    hs_ref[...] = h_ref[...] * dinv


def _tc_scale(h, dega, degb):
    n, m = h.shape
    return pl.pallas_call(
        _scale_body,
        grid=(n // ROW_BLK,),
        in_specs=[
            pl.BlockSpec((ROW_BLK, m), lambda i: (i, 0)),
            pl.BlockSpec((ROW_BLK, 1), lambda i: (i, 0)),
            pl.BlockSpec((ROW_BLK, 1), lambda i: (i, 0)),
        ],
        out_specs=[
            pl.BlockSpec((ROW_BLK, m), lambda i: (i, 0)),
            pl.BlockSpec((ROW_BLK, 1), lambda i: (i, 0)),
        ],
        out_shape=[
            jax.ShapeDtypeStruct((n, m), jnp.float32),
            jax.ShapeDtypeStruct((n, 1), jnp.float32),
        ],
    )(h, dega, degb)


def _mid_body(agga_ref, aggb_ref, hs1_ref, dinv_ref, b1_ref, w2_ref, hs2_ref):
    agg = jnp.concatenate([agga_ref[...], aggb_ref[...]], axis=-1)
    out1 = jax.nn.relu(dinv_ref[...] * (agg + hs1_ref[...]) + b1_ref[...])
    h2 = jnp.dot(out1, w2_ref[...], preferred_element_type=jnp.float32)
    hs2_ref[...] = h2 * dinv_ref[...]


def _tc_mid(agga, aggb, hs1, dinv, b1, W2):
    n = agga.shape[0]
    return pl.pallas_call(
        _mid_body,
        grid=(n // ROW_BLK,),
        in_specs=[
            pl.BlockSpec((ROW_BLK, 128), lambda i: (i, 0)),
            pl.BlockSpec((ROW_BLK, 128), lambda i: (i, 0)),
            pl.BlockSpec((ROW_BLK, D_H), lambda i: (i, 0)),
            pl.BlockSpec((ROW_BLK, 1), lambda i: (i, 0)),
            pl.BlockSpec((1, D_H), lambda i: (0, 0)),
            pl.BlockSpec((D_H, D_OUT), lambda i: (0, 0)),
        ],
        out_specs=pl.BlockSpec((ROW_BLK, D_OUT), lambda i: (i, 0)),
        out_shape=jax.ShapeDtypeStruct((n, D_OUT), jnp.float32),
    )(agga, aggb, hs1, dinv, b1, W2)


def _z_body(p0_ref, p1_ref, hs2_ref, dinv_ref, b2_ref, z_ref):
    z_ref[...] = (dinv_ref[...] * (p0_ref[...] + p1_ref[...] + hs2_ref[...])
                  + b2_ref[...])


def _tc_z(p0, p1, hs2, dinv, b2):
    n = p0.shape[0]
    return pl.pallas_call(
        _z_body,
        grid=(n // ROW_BLK,),
        in_specs=[
            pl.BlockSpec((ROW_BLK, D_OUT), lambda i: (i, 0)),
            pl.BlockSpec((ROW_BLK, D_OUT), lambda i: (i, 0)),
            pl.BlockSpec((ROW_BLK, D_OUT), lambda i: (i, 0)),
            pl.BlockSpec((ROW_BLK, 1), lambda i: (i, 0)),
            pl.BlockSpec((1, D_OUT), lambda i: (0, 0)),
        ],
        out_specs=pl.BlockSpec((ROW_BLK, D_OUT), lambda i: (i, 0)),
        out_shape=jax.ShapeDtypeStruct((n, D_OUT), jnp.float32),
    )(p0, p1, hs2, dinv, b2)


def _dot_body(zs_ref, zd_ref, o_ref):
    o_ref[...] = jnp.sum(zs_ref[...] * zd_ref[...], axis=-1, keepdims=True)


def _tc_dot(zs, zd):
    n = zs.shape[0]
    return pl.pallas_call(
        _dot_body,
        grid=(n // ROW_BLK,),
        in_specs=[
            pl.BlockSpec((ROW_BLK, D_OUT), lambda i: (i, 0)),
            pl.BlockSpec((ROW_BLK, D_OUT), lambda i: (i, 0)),
        ],
        out_specs=pl.BlockSpec((ROW_BLK, 1), lambda i: (i, 0)),
        out_shape=jax.ShapeDtypeStruct((n, 1), jnp.float32),
    )(zs, zd)


def _pad_rows(a):
    return jnp.concatenate(
        [a, jnp.zeros((NP - a.shape[0], a.shape[1]), a.dtype)], axis=0)


def kernel(x, edge_index, edge_label_index, W1, b1, W2, b2):
    epad = jnp.full((EP - E,), PAD_ROW, jnp.int32)
    src_p = jnp.concatenate([edge_index[0], epad])
    dst_p = jnp.concatenate([edge_index[1], epad])
    z128 = jnp.zeros((NP, 128), jnp.float32)

    degacc = _deg_call(dst_p, z128, jnp.ones((W, 128), jnp.float32))
    dega = degacc[0, :N, :1] + 1.0
    degb = degacc[1, :N, :1]

    h1 = _tc_matmul(x, W1)
    hs1, dinv = _tc_scale(h1, dega, degb)

    hs1p = _pad_rows(hs1[:, :128])
    hs1q = _pad_rows(hs1[:, 128:])
    agg_a, agg_b = _agg1_call(hs1p, hs1q, src_p, dst_p, z128)

    hs2 = _tc_mid(agg_a[:N], agg_b[:N], hs1, dinv, b1[None, :], W2)

    hs2p = _pad_rows(hs2)
    agg2 = _agg2_call(hs2p, src_p, dst_p, z128)

    z = _tc_z(agg2[0, :N], agg2[1, :N], hs2, dinv, b2[None, :])

    zp = _pad_rows(z)
    lpad = jnp.full((ELP - 2 * EL,), PAD_ROW, jnp.int32)
    dec_idx = jnp.concatenate([edge_label_index[0], edge_label_index[1], lpad])
    rows = _dec_call(zp, dec_idx)

    return _tc_dot(rows[:EL], rows[EL:2 * EL])[:, 0]
